# baseline (speedup 1.0000x reference)
import functools

import jax
import jax.numpy as jnp
from jax import lax
from jax.experimental import pallas as pl
from jax.experimental.pallas import tpu as pltpu
from jax.experimental.pallas import tpu_sc as plsc

DIM = 768
GRID_DIM = 24.0
SPAD = 592
NV = DIM // 16


def _score_body(ids_ref, x_ref, wq_ref, bq_ref, wk_ref, bk_ref,
                avg_ref, std_ref, eps_ref, w_ref, idx_ref):
    S = x_ref.shape[1]
    xb = x_ref[0]
    q = jnp.dot(xb, wq_ref[...], preferred_element_type=jnp.float32) + bq_ref[...]
    k = jnp.dot(xb, wk_ref[...], preferred_element_type=jnp.float32) + bk_ref[...]
    AT = jax.lax.dot_general(k, q, (((1,), (1,)), ((), ())),
                             preferred_element_type=jnp.float32)

    key_x = (eps_ref[0, :, 0:1] - avg_ref[0, 0:1, :]) / std_ref[0, 0:1, :]
    key_y = (eps_ref[0, :, 1:2] - avg_ref[0, 1:2, :]) / std_ref[0, 1:2, :]
    kx1, kx2 = jnp.ceil(key_x), jnp.floor(key_x)
    ky1, ky2 = jnp.ceil(key_y), jnp.floor(key_y)
    zero = jnp.zeros((1, 1), jnp.int32)
    idxs = []
    for fy, fx in ((ky1, kx1), (ky1, kx2), (ky2, kx1), (ky2, kx2)):
        ij = (GRID_DIM * fy + fx).astype(jnp.int32) % S
        idxs.append(jnp.concatenate([zero, ij], axis=1))

    rows = jax.lax.broadcasted_iota(jnp.int32, (S, S), 0)
    ats = []
    for j in range(4):
        mj = (rows == idxs[j]).astype(jnp.float32)
        ats.append(jnp.sum(AT * mj, axis=0, keepdims=True))
    at = jnp.concatenate(ats, axis=0)
    m = jnp.max(at, axis=0, keepdims=True)
    e = jnp.exp(at - m)
    w = e / jnp.sum(e, axis=0, keepdims=True)
    wpad = jnp.zeros((4, SPAD - S), jnp.float32)
    w_ref[0] = jnp.concatenate([w, wpad], axis=1)
    icat = jnp.concatenate(idxs, axis=0)
    ipad = jnp.broadcast_to(icat[:, S - 1:S], (4, SPAD - S))
    idx_ref[0] = jnp.concatenate([icat, ipad], axis=1)


def _value_body(y_ref, wv_ref, bv_ref, out_ref):
    S = y_ref.shape[1]
    out = jnp.dot(y_ref[0], wv_ref[...],
                  preferred_element_type=jnp.float32) + bv_ref[...]
    rows = jax.lax.broadcasted_iota(jnp.int32, (S, DIM), 0)
    out_ref[0] = jnp.where(rows == 0, 1.0, out)


def _make_sc_combine(B, S, D):
    n_chunks = SPAD // 16
    mesh = plsc.VectorSubcoreMesh(core_axis_name="c", subcore_axis_name="s")

    @functools.partial(
        pl.kernel, mesh=mesh,
        out_type=jax.ShapeDtypeStruct((B, S, D), jnp.float32),
        scratch_types=[
            pltpu.VMEM((4 * SPAD,), jnp.int32),
            pltpu.VMEM((SPAD * 64,), jnp.float32),
            pltpu.VMEM((64,), jnp.int32),
            pltpu.VMEM((64, DIM), jnp.float32),
            pltpu.VMEM((16, DIM), jnp.float32),
            pltpu.SemaphoreType.DMA,
            pltpu.SemaphoreType.DMA,
        ],
    )
    def sc_combine(xflat_hbm, idx_hbm, w_hbm, y_hbm,
                   idx_all, w_all, idx_v, rows_v, acc_v, sem, wsem):
        b = lax.axis_index("s") * 2 + lax.axis_index("c")
        pltpu.sync_copy(idx_hbm.at[b], idx_all)
        pltpu.async_copy(w_hbm.at[b], w_all, wsem)
        base = b * S

        cnt = None
        mjs = []
        for j in range(4):
            mx = idx_all[pl.ds(j * SPAD, 16)]
            mn = mx
            for t in range(1, SPAD // 16):
                v = idx_all[pl.ds(j * SPAD + t * 16, 16)]
                mx = jnp.maximum(mx, v)
                mn = jnp.minimum(mn, v)
            mjs.append(mx)
            dj = jnp.abs(mx - mn) + jnp.abs(mx - jnp.full((16,), mx[0]))
            cnt = dj if cnt is None else cnt + dj
        total = cnt[0]
        for l in range(1, 16):
            total = total + cnt[l]
        uni = total == 0
        pltpu.make_async_copy(w_hbm.at[b], w_all, wsem).wait()

        @pl.when(uni)
        def _fast():
            io = lax.iota(jnp.int32, 16)
            sel = jnp.where(io == 0, mjs[0],
                            jnp.where(io == 1, mjs[1],
                                      jnp.where(io == 2, mjs[2], mjs[3])))
            idx_v[pl.ds(0, 16)] = sel + base
            pltpu.async_copy(xflat_hbm.at[idx_v.at[pl.ds(0, 16)]],
                             rows_v.at[pl.ds(0, 16)], sem).wait()

            def chunk(g, _):
                for h in range(2):
                    wbase = g * 1024 + h * 512
                    ws = [[w_all[pl.ds(wbase + k * 64 + j * 16, 16)]
                           for j in range(4)] for k in range(8)]

                    def col(i, _):
                        r = [rows_v[j, pl.ds(i * 16, 16)] for j in range(4)]
                        for k in range(8):
                            m0 = ws[k][0] * r[0]
                            m1 = ws[k][1] * r[1]
                            m2 = ws[k][2] * r[2]
                            m3 = ws[k][3] * r[3]
                            acc_v[h * 8 + k, pl.ds(i * 16, 16)] = (
                                (m0 + m1) + (m2 + m3))
                        return _

                    lax.fori_loop(0, NV, col, None, unroll=2)

                @pl.when(g < SPAD // 16 - 1)
                def _():
                    pltpu.sync_copy(acc_v, y_hbm.at[b, pl.ds(g * 16, 16)])

                @pl.when(g == SPAD // 16 - 1)
                def _():
                    pltpu.sync_copy(acc_v.at[pl.ds(0, 1)],
                                    y_hbm.at[b, pl.ds(g * 16, 1)])
                return _

            lax.fori_loop(0, SPAD // 16, chunk, None)

        @pl.when(jnp.logical_not(uni))
        def _general():
            def chunk(g, _):
                s0 = g * 16
                for j in range(4):
                    idx_v[pl.ds(j * 16, 16)] = (
                        idx_all[pl.ds(j * SPAD + s0, 16)] + base)
                pltpu.async_copy(xflat_hbm.at[idx_v], rows_v, sem).wait()

                def pos(sl, _):
                    wb = s0 * 64 + sl * 64
                    ws = [w_all[pl.ds(wb + j * 16, 16)] for j in range(4)]
                    for i in range(NV):
                        m0 = ws[0] * rows_v[sl, pl.ds(i * 16, 16)]
                        m1 = ws[1] * rows_v[16 + sl, pl.ds(i * 16, 16)]
                        m2 = ws[2] * rows_v[32 + sl, pl.ds(i * 16, 16)]
                        m3 = ws[3] * rows_v[48 + sl, pl.ds(i * 16, 16)]
                        acc_v[sl, pl.ds(i * 16, 16)] = (m0 + m1) + (m2 + m3)
                    return _

                lax.fori_loop(0, 16, pos, None, unroll=4)

                @pl.when(g < SPAD // 16 - 1)
                def _():
                    pltpu.sync_copy(acc_v, y_hbm.at[b, pl.ds(s0, 16)])

                @pl.when(g == SPAD // 16 - 1)
                def _():
                    pltpu.sync_copy(acc_v.at[pl.ds(0, 1)],
                                    y_hbm.at[b, pl.ds(s0, 1)])
                return _

            lax.fori_loop(0, SPAD // 16, chunk, None)

    return sc_combine


def kernel(x, img_ids, mask, Wq, bq, Wk, bk, Wv, bv, avgs, std_devs):
    B, S, D = x.shape
    P = S - 1
    eps = jax.random.normal(jax.random.key(1234), (B, 2), dtype=jnp.float32)

    grid_spec = pltpu.PrefetchScalarGridSpec(
        num_scalar_prefetch=1,
        grid=(B,),
        in_specs=[
            pl.BlockSpec((1, S, D), lambda b, ids: (b, 0, 0)),
            pl.BlockSpec((D, D), lambda b, ids: (0, 0)),
            pl.BlockSpec((1, D), lambda b, ids: (0, 0)),
            pl.BlockSpec((D, D), lambda b, ids: (0, 0)),
            pl.BlockSpec((1, D), lambda b, ids: (0, 0)),
            pl.BlockSpec((1, 2, P), lambda b, ids: (ids[b], 0, 0)),
            pl.BlockSpec((1, 2, P), lambda b, ids: (ids[b], 0, 0)),
            pl.BlockSpec((1, 1, 2), lambda b, ids: (b, 0, 0)),
        ],
        out_specs=[
            pl.BlockSpec((1, 4, SPAD), lambda b, ids: (b, 0, 0)),
            pl.BlockSpec((1, 4, SPAD), lambda b, ids: (b, 0, 0)),
        ],
    )
    w, idx = pl.pallas_call(
        _score_body,
        grid_spec=grid_spec,
        out_shape=[
            jax.ShapeDtypeStruct((B, 4, SPAD), jnp.float32),
            jax.ShapeDtypeStruct((B, 4, SPAD), jnp.int32),
        ],
    )(img_ids, x, Wq, bq.reshape(1, D), Wk, bk.reshape(1, D),
      avgs, std_devs, eps.reshape(B, 1, 2))

    wexp = jnp.broadcast_to(
        jnp.transpose(w, (0, 2, 1))[:, :, :, None], (B, SPAD, 4, 16)
    ).reshape(B, SPAD * 64)
    sc_combine = _make_sc_combine(B, S, D)
    y = sc_combine(x.reshape(B * S, D), idx.reshape(B, 4 * SPAD), wexp)

    out = pl.pallas_call(
        _value_body,
        grid=(B,),
        in_specs=[
            pl.BlockSpec((1, S, D), lambda b: (b, 0, 0)),
            pl.BlockSpec((D, D), lambda b: (0, 0)),
            pl.BlockSpec((1, D), lambda b: (0, 0)),
        ],
        out_specs=pl.BlockSpec((1, S, D), lambda b: (b, 0, 0)),
        out_shape=jax.ShapeDtypeStruct((B, S, D), jnp.float32),
    )(y, Wv, bv.reshape(1, D))
    return out


# speedup vs baseline: 2.0194x; 2.0194x over previous
import functools

import jax
import jax.numpy as jnp
from jax import lax
from jax.experimental import pallas as pl
from jax.experimental.pallas import tpu as pltpu
from jax.experimental.pallas import tpu_sc as plsc

DIM = 768
GRID_DIM = 24.0
SPAD = 592
NV = DIM // 16


def _score_body(ids_ref, x_ref, wq_ref, bq_ref, wk_ref, bk_ref,
                avg_ref, std_ref, eps_ref, w_ref, idx_ref):
    S = x_ref.shape[1]
    xb = x_ref[0]
    q = jnp.dot(xb, wq_ref[...], preferred_element_type=jnp.float32) + bq_ref[...]
    k = jnp.dot(xb, wk_ref[...], preferred_element_type=jnp.float32) + bk_ref[...]
    AT = jax.lax.dot_general(k, q, (((1,), (1,)), ((), ())),
                             preferred_element_type=jnp.float32)

    key_x = (eps_ref[0, :, 0:1] - avg_ref[0, 0:1, :]) / std_ref[0, 0:1, :]
    key_y = (eps_ref[0, :, 1:2] - avg_ref[0, 1:2, :]) / std_ref[0, 1:2, :]
    kx1, kx2 = jnp.ceil(key_x), jnp.floor(key_x)
    ky1, ky2 = jnp.ceil(key_y), jnp.floor(key_y)
    idxs = []
    for fy, fx in ((ky1, kx1), (ky1, kx2), (ky2, kx1), (ky2, kx2)):
        ij = (GRID_DIM * fy + fx).astype(jnp.int32) % S
        idxs.append(jnp.concatenate([ij[:, :1], ij], axis=1))

    rows = jax.lax.broadcasted_iota(jnp.int32, (S, S), 0)
    ats = []
    for j in range(4):
        mj = (rows == idxs[j]).astype(jnp.float32)
        ats.append(jnp.sum(AT * mj, axis=0, keepdims=True))
    at = jnp.concatenate(ats, axis=0)
    m = jnp.max(at, axis=0, keepdims=True)
    e = jnp.exp(at - m)
    w = e / jnp.sum(e, axis=0, keepdims=True)
    wpad = jnp.zeros((4, SPAD - S), jnp.float32)
    w_ref[0] = jnp.concatenate([w, wpad], axis=1)
    icat = jnp.concatenate(idxs, axis=0)
    ipad = jnp.broadcast_to(icat[:, S - 1:S], (4, SPAD - S))
    idx_ref[0] = jnp.concatenate([icat, ipad], axis=1)


def _value_body(y_ref, wv_ref, bv_ref, out_ref):
    S = y_ref.shape[1]
    out = jnp.dot(y_ref[0], wv_ref[...],
                  preferred_element_type=jnp.float32) + bv_ref[...]
    rows = jax.lax.broadcasted_iota(jnp.int32, (S, DIM), 0)
    out_ref[0] = jnp.where(rows == 0, 1.0, out)


def _make_sc_combine(B, S, D):
    n_chunks = SPAD // 16
    mesh = plsc.VectorSubcoreMesh(core_axis_name="c", subcore_axis_name="s")

    @functools.partial(
        pl.kernel, mesh=mesh,
        out_type=jax.ShapeDtypeStruct((B, S, D), jnp.float32),
        scratch_types=[
            pltpu.VMEM((4 * SPAD,), jnp.int32),
            pltpu.VMEM((SPAD * 64,), jnp.float32),
            pltpu.VMEM((64,), jnp.int32),
            pltpu.VMEM((64, DIM), jnp.float32),
            pltpu.VMEM((16, DIM), jnp.float32),
            pltpu.SemaphoreType.DMA,
            pltpu.SemaphoreType.DMA,
        ],
    )
    def sc_combine(xflat_hbm, idx_hbm, w_hbm, y_hbm,
                   idx_all, w_all, idx_v, rows_v, acc_v, sem, wsem):
        b = lax.axis_index("s") * 2 + lax.axis_index("c")
        pltpu.sync_copy(idx_hbm.at[b], idx_all)
        pltpu.async_copy(w_hbm.at[b], w_all, wsem)
        base = b * S

        cnt = None
        mjs = []
        for j in range(4):
            mx = idx_all[pl.ds(j * SPAD, 16)]
            mn = mx
            for t in range(1, SPAD // 16):
                v = idx_all[pl.ds(j * SPAD + t * 16, 16)]
                mx = jnp.maximum(mx, v)
                mn = jnp.minimum(mn, v)
            mjs.append(mx)
            dj = jnp.abs(mx - mn) + jnp.abs(mx - jnp.full((16,), mx[0]))
            cnt = dj if cnt is None else cnt + dj
        total = cnt[0]
        for l in range(1, 16):
            total = total + cnt[l]
        uni = total == 0
        pltpu.make_async_copy(w_hbm.at[b], w_all, wsem).wait()

        @pl.when(uni)
        def _fast():
            io = lax.iota(jnp.int32, 16)
            sel = jnp.where(io == 0, mjs[0],
                            jnp.where(io == 1, mjs[1],
                                      jnp.where(io == 2, mjs[2], mjs[3])))
            idx_v[pl.ds(0, 16)] = sel + base
            pltpu.async_copy(xflat_hbm.at[idx_v.at[pl.ds(0, 16)]],
                             rows_v.at[pl.ds(0, 16)], sem).wait()

            def chunk(g, _):
                for h in range(2):
                    wbase = g * 1024 + h * 512
                    ws = [[w_all[pl.ds(wbase + k * 64 + j * 16, 16)]
                           for j in range(4)] for k in range(8)]

                    def col(i, _):
                        r = [rows_v[j, pl.ds(i * 16, 16)] for j in range(4)]
                        for k in range(8):
                            m0 = ws[k][0] * r[0]
                            m1 = ws[k][1] * r[1]
                            m2 = ws[k][2] * r[2]
                            m3 = ws[k][3] * r[3]
                            acc_v[h * 8 + k, pl.ds(i * 16, 16)] = (
                                (m0 + m1) + (m2 + m3))
                        return _

                    lax.fori_loop(0, NV, col, None, unroll=2)

                @pl.when(g < SPAD // 16 - 1)
                def _():
                    pltpu.sync_copy(acc_v, y_hbm.at[b, pl.ds(g * 16, 16)])

                @pl.when(g == SPAD // 16 - 1)
                def _():
                    pltpu.sync_copy(acc_v.at[pl.ds(0, 1)],
                                    y_hbm.at[b, pl.ds(g * 16, 1)])
                return _

            lax.fori_loop(0, SPAD // 16, chunk, None)

        @pl.when(jnp.logical_not(uni))
        def _general():
            def chunk(g, _):
                s0 = g * 16
                for j in range(4):
                    idx_v[pl.ds(j * 16, 16)] = (
                        idx_all[pl.ds(j * SPAD + s0, 16)] + base)
                pltpu.async_copy(xflat_hbm.at[idx_v], rows_v, sem).wait()

                def pos(sl, _):
                    wb = s0 * 64 + sl * 64
                    ws = [w_all[pl.ds(wb + j * 16, 16)] for j in range(4)]
                    for i in range(NV):
                        m0 = ws[0] * rows_v[sl, pl.ds(i * 16, 16)]
                        m1 = ws[1] * rows_v[16 + sl, pl.ds(i * 16, 16)]
                        m2 = ws[2] * rows_v[32 + sl, pl.ds(i * 16, 16)]
                        m3 = ws[3] * rows_v[48 + sl, pl.ds(i * 16, 16)]
                        acc_v[sl, pl.ds(i * 16, 16)] = (m0 + m1) + (m2 + m3)
                    return _

                lax.fori_loop(0, 16, pos, None, unroll=4)

                @pl.when(g < SPAD // 16 - 1)
                def _():
                    pltpu.sync_copy(acc_v, y_hbm.at[b, pl.ds(s0, 16)])

                @pl.when(g == SPAD // 16 - 1)
                def _():
                    pltpu.sync_copy(acc_v.at[pl.ds(0, 1)],
                                    y_hbm.at[b, pl.ds(s0, 1)])
                return _

            lax.fori_loop(0, SPAD // 16, chunk, None)

    return sc_combine


def kernel(x, img_ids, mask, Wq, bq, Wk, bk, Wv, bv, avgs, std_devs):
    B, S, D = x.shape
    P = S - 1
    eps = jax.random.normal(jax.random.key(1234), (B, 2), dtype=jnp.float32)

    grid_spec = pltpu.PrefetchScalarGridSpec(
        num_scalar_prefetch=1,
        grid=(B,),
        in_specs=[
            pl.BlockSpec((1, S, D), lambda b, ids: (b, 0, 0)),
            pl.BlockSpec((D, D), lambda b, ids: (0, 0)),
            pl.BlockSpec((1, D), lambda b, ids: (0, 0)),
            pl.BlockSpec((D, D), lambda b, ids: (0, 0)),
            pl.BlockSpec((1, D), lambda b, ids: (0, 0)),
            pl.BlockSpec((1, 2, P), lambda b, ids: (ids[b], 0, 0)),
            pl.BlockSpec((1, 2, P), lambda b, ids: (ids[b], 0, 0)),
            pl.BlockSpec((1, 1, 2), lambda b, ids: (b, 0, 0)),
        ],
        out_specs=[
            pl.BlockSpec((1, 4, SPAD), lambda b, ids: (b, 0, 0)),
            pl.BlockSpec((1, 4, SPAD), lambda b, ids: (b, 0, 0)),
        ],
    )
    w, idx = pl.pallas_call(
        _score_body,
        grid_spec=grid_spec,
        out_shape=[
            jax.ShapeDtypeStruct((B, 4, SPAD), jnp.float32),
            jax.ShapeDtypeStruct((B, 4, SPAD), jnp.int32),
        ],
    )(img_ids, x, Wq, bq.reshape(1, D), Wk, bk.reshape(1, D),
      avgs, std_devs, eps.reshape(B, 1, 2))

    wexp = jnp.broadcast_to(
        jnp.transpose(w, (0, 2, 1))[:, :, :, None], (B, SPAD, 4, 16)
    ).reshape(B, SPAD * 64)
    sc_combine = _make_sc_combine(B, S, D)
    y = sc_combine(x.reshape(B * S, D), idx.reshape(B, 4 * SPAD), wexp)

    out = pl.pallas_call(
        _value_body,
        grid=(B,),
        in_specs=[
            pl.BlockSpec((1, S, D), lambda b: (b, 0, 0)),
            pl.BlockSpec((D, D), lambda b: (0, 0)),
            pl.BlockSpec((1, D), lambda b: (0, 0)),
        ],
        out_specs=pl.BlockSpec((1, S, D), lambda b: (b, 0, 0)),
        out_shape=jax.ShapeDtypeStruct((B, S, D), jnp.float32),
    )(y, Wv, bv.reshape(1, D))
    return out
